# baseline (speedup 1.0000x reference)
import functools

import jax
import jax.numpy as jnp
from jax import lax
from jax.experimental import pallas as pl
from jax.experimental.pallas import tpu as pltpu
from jax.experimental.pallas import tpu_sc as plsc

N_NODES = 10000
N_EDGES = 320000
NC = 2
NS = 16
NW = NC * NS
CH = 32
SLOTS = 8
E_PAD = 327680
N_CHUNKS_P = E_PAD // CH
CPW = N_CHUNKS_P // NW
PHASES = 8
PPC = CPW // PHASES
PAD_E = E_PAD - N_EDGES
N_TAB = N_NODES + 496
ROWS_MAIN = 624
ROWS_LAST = N_NODES - 15 * ROWS_MAIN

ROW_BLK = 2000


def _mm1_body(f_ref, w_ref, n_ref, o_ref):
    t = jnp.dot(f_ref[...], w_ref[...], preferred_element_type=jnp.float32)
    o_ref[...] = t * n_ref[...]


def _mm1(features, W, norm):
    return pl.pallas_call(
        _mm1_body,
        grid=(N_NODES // ROW_BLK,),
        in_specs=[
            pl.BlockSpec((ROW_BLK, features.shape[1]), lambda i: (i, 0)),
            pl.BlockSpec(W.shape, lambda i: (0, 0)),
            pl.BlockSpec((ROW_BLK, 1), lambda i: (i, 0)),
        ],
        out_specs=pl.BlockSpec((ROW_BLK, W.shape[1]), lambda i: (i, 0)),
        out_shape=jax.ShapeDtypeStruct((N_NODES, W.shape[1]), jnp.float32),
    )(features, W, norm)


def _mm2_body(p_ref, b_ref, w_ref, n_ref, o_ref):
    h = p_ref[0] + p_ref[1] + b_ref[...]
    h = jnp.maximum(h, 0.0)
    t = jnp.dot(h, w_ref[...], preferred_element_type=jnp.float32)
    o_ref[...] = t * n_ref[...]


def _mm2(partials, b, W, norm):
    d_in = partials.shape[2]
    return pl.pallas_call(
        _mm2_body,
        grid=(N_NODES // ROW_BLK,),
        in_specs=[
            pl.BlockSpec((2, ROW_BLK, d_in), lambda i: (0, i, 0)),
            pl.BlockSpec((1, d_in), lambda i: (0, 0)),
            pl.BlockSpec(W.shape, lambda i: (0, 0)),
            pl.BlockSpec((ROW_BLK, 1), lambda i: (i, 0)),
        ],
        out_specs=pl.BlockSpec((ROW_BLK, W.shape[1]), lambda i: (i, 0)),
        out_shape=jax.ShapeDtypeStruct((N_NODES, W.shape[1]), jnp.float32),
    )(partials, b.reshape(1, d_in), W, norm)


def _scale_body(p_ref, b_ref, n_ref, o_ref):
    h = p_ref[0] + p_ref[1] + b_ref[...]
    o_ref[...] = jnp.maximum(h, 0.0) * n_ref[...]


def _scale(partials, b, norm):
    d = partials.shape[2]
    return pl.pallas_call(
        _scale_body,
        grid=(N_NODES // ROW_BLK,),
        in_specs=[
            pl.BlockSpec((2, ROW_BLK, d), lambda i: (0, i, 0)),
            pl.BlockSpec((1, d), lambda i: (0, 0)),
            pl.BlockSpec((ROW_BLK, 1), lambda i: (i, 0)),
        ],
        out_specs=pl.BlockSpec((ROW_BLK, d), lambda i: (i, 0)),
        out_shape=jax.ShapeDtypeStruct((N_NODES, d), jnp.float32),
    )(partials, b.reshape(1, d), norm)


def _final_body(p_ref, w_ref, b_ref, o_ref):
    h = p_ref[0] + p_ref[1]
    o_ref[...] = (
        jnp.dot(h, w_ref[...], preferred_element_type=jnp.float32) + b_ref[...]
    )


def _final(partials, W, b):
    d_in = partials.shape[2]
    d_out = W.shape[1]
    return pl.pallas_call(
        _final_body,
        grid=(N_NODES // ROW_BLK,),
        in_specs=[
            pl.BlockSpec((2, ROW_BLK, d_in), lambda i: (0, i, 0)),
            pl.BlockSpec(W.shape, lambda i: (0, 0)),
            pl.BlockSpec((1, d_out), lambda i: (0, 0)),
        ],
        out_specs=pl.BlockSpec((ROW_BLK, d_out), lambda i: (i, 0)),
        out_shape=jax.ShapeDtypeStruct((N_NODES, d_out), jnp.float32),
    )(partials, W, b.reshape(1, d_out))


@functools.lru_cache(maxsize=None)
def _make_agg(feat):
    mesh = plsc.VectorSubcoreMesh(core_axis_name="c", subcore_axis_name="s")

    @functools.partial(
        pl.kernel,
        out_type=jax.ShapeDtypeStruct((NC, N_NODES, feat), jnp.float32),
        mesh=mesh,
        scratch_types=(
            [
                pltpu.VMEM((PPC * CH,), jnp.int32),
                pltpu.VMEM((PPC, CH), jnp.int32),
            ]
            + [pltpu.VMEM((CH, feat), jnp.float32) for _ in range(SLOTS)]
            + [pltpu.VMEM_SHARED((N_NODES, feat), jnp.float32)]
            + [pltpu.SemaphoreType.DMA for _ in range(2 * SLOTS)]
        ),
    )
    def agg(t_hbm, src_hbm, dst_hbm, zero_hbm, out_hbm,
            idx_s, idx_d, *rest):
        rows = rest[:SLOTS]
        accum = rest[SLOTS]
        gsem = rest[SLOTS + 1:2 * SLOTS + 1]
        ssem = rest[2 * SLOTS + 1:]
        c = lax.axis_index("c")
        s = lax.axis_index("s")
        wid = s * NC + c
        row0 = pl.multiple_of(s * ROWS_MAIN, 8)
        @pl.when(s < NS - 1)
        def _():
            pltpu.sync_copy(zero_hbm.at[pl.ds(0, ROWS_MAIN)],
                            accum.at[pl.ds(row0, ROWS_MAIN)])

        @pl.when(s == NS - 1)
        def _():
            pltpu.sync_copy(zero_hbm, accum.at[pl.ds(row0, ROWS_LAST)])

        plsc.subcore_barrier()

        def fire_gather(k, sl):
            off = k * CH
            if not isinstance(off, int):
                off = pl.multiple_of(off, 8)
            pltpu.async_copy(t_hbm.at[idx_s.at[pl.ds(off, CH)]],
                             rows[sl], gsem[sl])

        def fire_scatter(k, sl):
            pltpu.async_copy(rows[sl], accum.at[idx_d.at[k]], ssem[sl],
                             add=True)

        def drain(sem):
            pltpu.make_async_copy(t_hbm.at[pl.ds(0, CH)], rows[0], sem).wait()

        for h in range(PHASES):
            c0 = wid * CPW + h * PPC
            pltpu.sync_copy(
                src_hbm.at[pl.ds(pl.multiple_of(c0 * CH, 8), PPC * CH)],
                idx_s)
            pltpu.sync_copy(
                dst_hbm.at[pl.ds(pl.multiple_of(c0, 8), PPC)], idx_d)

            for b in range(SLOTS):
                fire_gather(b, b)
            for b in range(SLOTS):
                drain(gsem[b])
                fire_scatter(b, b)

            @pl.loop(1, PPC // SLOTS)
            def _(j):
                k0 = j * SLOTS
                for b in range(SLOTS):
                    drain(ssem[b])
                    fire_gather(k0 + b, b)
                for b in range(SLOTS):
                    drain(gsem[b])
                    fire_scatter(k0 + b, b)

            for b in range(SLOTS):
                drain(ssem[b])

        plsc.subcore_barrier()

        @pl.when(s < NS - 1)
        def _():
            pltpu.sync_copy(accum.at[pl.ds(row0, ROWS_MAIN)],
                            out_hbm.at[c, pl.ds(row0, ROWS_MAIN)])

        @pl.when(s == NS - 1)
        def _():
            pltpu.sync_copy(accum.at[pl.ds(row0, ROWS_LAST)],
                            out_hbm.at[c, pl.ds(row0, ROWS_LAST)])

    return agg


def _agg(t, src, dst):
    feat = t.shape[1]
    zero = jnp.zeros((ROWS_LAST, feat), jnp.float32)
    tp = jnp.concatenate([t, jnp.zeros((N_TAB - N_NODES, feat), t.dtype)])
    srcp = jnp.concatenate(
        [src, N_NODES + (jnp.arange(PAD_E, dtype=jnp.int32) % 496)])
    dstp = jnp.concatenate(
        [dst, (jnp.arange(PAD_E, dtype=jnp.int32) % 625) * 16]
    ).reshape(N_CHUNKS_P, CH)
    return _make_agg(feat)(tp, srcp, dstp, zero)


def kernel(features, edge_index, norm, W1, b1, W2, b2, W3, b3):
    src = edge_index[0]
    dst = edge_index[1]
    t1 = _mm1(features, W1, norm)
    p1 = _agg(t1, src, dst)
    t2 = _mm2(p1, b1, W2, norm)
    p2 = _agg(t2, src, dst)
    t3 = _scale(p2, b2, norm)
    p3 = _agg(t3, src, dst)
    return _final(p3, W3, b3)


# speedup vs baseline: 1.0720x; 1.0720x over previous
import functools

import jax
import jax.numpy as jnp
from jax import lax
from jax.experimental import pallas as pl
from jax.experimental.pallas import tpu as pltpu
from jax.experimental.pallas import tpu_sc as plsc

N_NODES = 10000
N_EDGES = 320000
NC = 2
NS = 16
NW = NC * NS
CH = 64
SLOTS = 4
E_PAD = 327680
N_CHUNKS_P = E_PAD // CH
CPW = N_CHUNKS_P // NW
PHASES = 2
PPC = CPW // PHASES
PAD_E = E_PAD - N_EDGES
N_TAB = N_NODES + 496
ROWS_MAIN = 624
ROWS_LAST = N_NODES - 15 * ROWS_MAIN

ROW_BLK = 2000


def _mm1_body(f_ref, w_ref, n_ref, o_ref):
    t = jnp.dot(f_ref[...], w_ref[...], preferred_element_type=jnp.float32)
    o_ref[...] = t * n_ref[...]


def _mm1(features, W, norm):
    return pl.pallas_call(
        _mm1_body,
        grid=(N_NODES // ROW_BLK,),
        in_specs=[
            pl.BlockSpec((ROW_BLK, features.shape[1]), lambda i: (i, 0)),
            pl.BlockSpec(W.shape, lambda i: (0, 0)),
            pl.BlockSpec((ROW_BLK, 1), lambda i: (i, 0)),
        ],
        out_specs=pl.BlockSpec((ROW_BLK, W.shape[1]), lambda i: (i, 0)),
        out_shape=jax.ShapeDtypeStruct((N_NODES, W.shape[1]), jnp.float32),
    )(features, W, norm)


def _mm2_body(p_ref, b_ref, w_ref, n_ref, o_ref):
    h = p_ref[0] + p_ref[1] + b_ref[...]
    h = jnp.maximum(h, 0.0)
    t = jnp.dot(h, w_ref[...], preferred_element_type=jnp.float32)
    o_ref[...] = t * n_ref[...]


def _mm2(partials, b, W, norm):
    d_in = partials.shape[2]
    return pl.pallas_call(
        _mm2_body,
        grid=(N_NODES // ROW_BLK,),
        in_specs=[
            pl.BlockSpec((2, ROW_BLK, d_in), lambda i: (0, i, 0)),
            pl.BlockSpec((1, d_in), lambda i: (0, 0)),
            pl.BlockSpec(W.shape, lambda i: (0, 0)),
            pl.BlockSpec((ROW_BLK, 1), lambda i: (i, 0)),
        ],
        out_specs=pl.BlockSpec((ROW_BLK, W.shape[1]), lambda i: (i, 0)),
        out_shape=jax.ShapeDtypeStruct((N_NODES, W.shape[1]), jnp.float32),
    )(partials, b.reshape(1, d_in), W, norm)


def _scale_body(p_ref, b_ref, n_ref, o_ref):
    h = p_ref[0] + p_ref[1] + b_ref[...]
    o_ref[...] = jnp.maximum(h, 0.0) * n_ref[...]


def _scale(partials, b, norm):
    d = partials.shape[2]
    return pl.pallas_call(
        _scale_body,
        grid=(N_NODES // ROW_BLK,),
        in_specs=[
            pl.BlockSpec((2, ROW_BLK, d), lambda i: (0, i, 0)),
            pl.BlockSpec((1, d), lambda i: (0, 0)),
            pl.BlockSpec((ROW_BLK, 1), lambda i: (i, 0)),
        ],
        out_specs=pl.BlockSpec((ROW_BLK, d), lambda i: (i, 0)),
        out_shape=jax.ShapeDtypeStruct((N_NODES, d), jnp.float32),
    )(partials, b.reshape(1, d), norm)


def _final_body(p_ref, w_ref, b_ref, o_ref):
    h = p_ref[0] + p_ref[1]
    o_ref[...] = (
        jnp.dot(h, w_ref[...], preferred_element_type=jnp.float32) + b_ref[...]
    )


def _final(partials, W, b):
    d_in = partials.shape[2]
    d_out = W.shape[1]
    return pl.pallas_call(
        _final_body,
        grid=(N_NODES // ROW_BLK,),
        in_specs=[
            pl.BlockSpec((2, ROW_BLK, d_in), lambda i: (0, i, 0)),
            pl.BlockSpec(W.shape, lambda i: (0, 0)),
            pl.BlockSpec((1, d_out), lambda i: (0, 0)),
        ],
        out_specs=pl.BlockSpec((ROW_BLK, d_out), lambda i: (i, 0)),
        out_shape=jax.ShapeDtypeStruct((N_NODES, d_out), jnp.float32),
    )(partials, W, b.reshape(1, d_out))


@functools.lru_cache(maxsize=None)
def _make_agg(feat):
    mesh = plsc.VectorSubcoreMesh(core_axis_name="c", subcore_axis_name="s")

    @functools.partial(
        pl.kernel,
        out_type=jax.ShapeDtypeStruct((NC, N_NODES, feat), jnp.float32),
        mesh=mesh,
        scratch_types=(
            [
                pltpu.VMEM((PPC * CH,), jnp.int32),
                pltpu.VMEM((PPC, CH), jnp.int32),
            ]
            + [pltpu.VMEM((CH, feat), jnp.float32) for _ in range(SLOTS)]
            + [pltpu.VMEM_SHARED((N_NODES, feat), jnp.float32)]
            + [pltpu.SemaphoreType.DMA for _ in range(2 * SLOTS)]
        ),
    )
    def agg(t_hbm, src_hbm, dst_hbm, zero_hbm, out_hbm,
            idx_s, idx_d, *rest):
        rows = rest[:SLOTS]
        accum = rest[SLOTS]
        gsem = rest[SLOTS + 1:2 * SLOTS + 1]
        ssem = rest[2 * SLOTS + 1:]
        c = lax.axis_index("c")
        s = lax.axis_index("s")
        wid = s * NC + c
        row0 = pl.multiple_of(s * ROWS_MAIN, 8)
        @pl.when(s < NS - 1)
        def _():
            pltpu.sync_copy(zero_hbm.at[pl.ds(0, ROWS_MAIN)],
                            accum.at[pl.ds(row0, ROWS_MAIN)])

        @pl.when(s == NS - 1)
        def _():
            pltpu.sync_copy(zero_hbm, accum.at[pl.ds(row0, ROWS_LAST)])

        plsc.subcore_barrier()

        def fire_gather(k, sl):
            off = k * CH
            if not isinstance(off, int):
                off = pl.multiple_of(off, 8)
            pltpu.async_copy(t_hbm.at[idx_s.at[pl.ds(off, CH)]],
                             rows[sl], gsem[sl])

        def fire_scatter(k, sl):
            pltpu.async_copy(rows[sl], accum.at[idx_d.at[k]], ssem[sl],
                             add=True)

        def drain(sem):
            pltpu.make_async_copy(t_hbm.at[pl.ds(0, CH)], rows[0], sem).wait()

        for h in range(PHASES):
            c0 = wid * CPW + h * PPC
            pltpu.sync_copy(
                src_hbm.at[pl.ds(pl.multiple_of(c0 * CH, 8), PPC * CH)],
                idx_s)
            pltpu.sync_copy(
                dst_hbm.at[pl.ds(pl.multiple_of(c0, 8), PPC)], idx_d)

            for b in range(SLOTS):
                fire_gather(b, b)
            for b in range(SLOTS):
                drain(gsem[b])
                fire_scatter(b, b)

            @pl.loop(1, PPC // SLOTS)
            def _(j):
                k0 = j * SLOTS
                for b in range(SLOTS):
                    drain(ssem[b])
                    fire_gather(k0 + b, b)
                for b in range(SLOTS):
                    drain(gsem[b])
                    fire_scatter(k0 + b, b)

            for b in range(SLOTS):
                drain(ssem[b])

        plsc.subcore_barrier()

        @pl.when(s < NS - 1)
        def _():
            pltpu.sync_copy(accum.at[pl.ds(row0, ROWS_MAIN)],
                            out_hbm.at[c, pl.ds(row0, ROWS_MAIN)])

        @pl.when(s == NS - 1)
        def _():
            pltpu.sync_copy(accum.at[pl.ds(row0, ROWS_LAST)],
                            out_hbm.at[c, pl.ds(row0, ROWS_LAST)])

    return agg


def _agg(t, src, dst):
    feat = t.shape[1]
    zero = jnp.zeros((ROWS_LAST, feat), jnp.float32)
    tp = jnp.concatenate([t, jnp.zeros((N_TAB - N_NODES, feat), t.dtype)])
    srcp = jnp.concatenate(
        [src, N_NODES + (jnp.arange(PAD_E, dtype=jnp.int32) % 496)])
    dstp = jnp.concatenate(
        [dst, (jnp.arange(PAD_E, dtype=jnp.int32) % 625) * 16]
    ).reshape(N_CHUNKS_P, CH)
    return _make_agg(feat)(tp, srcp, dstp, zero)


def kernel(features, edge_index, norm, W1, b1, W2, b2, W3, b3):
    src = edge_index[0]
    dst = edge_index[1]
    t1 = _mm1(features, W1, norm)
    p1 = _agg(t1, src, dst)
    t2 = _mm2(p1, b1, W2, norm)
    p2 = _agg(t2, src, dst)
    t3 = _scale(p2, b2, norm)
    p3 = _agg(t3, src, dst)
    return _final(p3, W3, b3)
